# baseline (speedup 1.0000x reference)
import jax
import jax.numpy as jnp
from jax import lax
from jax.experimental import pallas as pl
from jax.experimental.pallas import tpu as pltpu

D = 256
KEV = 100000
NBLK = 25
RB = 4000
PB = 4096
PADTOT = NBLK * PB
NEG = float("-inf")


def _score_body(claim_ref, WqT_ref, bq_ref, WkT_ref, bk_ref, keys_ref, out_ref):
    c0 = claim_ref[0:1, :]
    qr = jnp.dot(c0, WqT_ref[...], preferred_element_type=jnp.float32)
    qr = qr + bq_ref[...][None, :]
    qn = qr / jnp.maximum(jnp.sqrt(jnp.sum(qr * qr)), 1e-12)
    z = jnp.dot(keys_ref[...], WkT_ref[...], preferred_element_type=jnp.float32)
    z = z + bk_ref[...][None, :]
    num = lax.dot_general(z, qn, (((1,), (1,)), ((), ())),
                          preferred_element_type=jnp.float32)
    ones = (lax.broadcasted_iota(jnp.int32, (D, 1), 0) >= 0).astype(jnp.float32)
    d2 = jnp.dot(z * z, ones, preferred_element_type=jnp.float32)
    s = (num / jnp.maximum(jnp.sqrt(d2), 1e-12)).reshape(RB)
    out_ref[...] = jnp.concatenate(
        [s, jnp.full((PB - RB,), NEG, jnp.float32)], axis=0)


def _select_body(scores_ref, values_ref, claim_ref, V1_ref, c1_ref, V2_ref,
                 c2_ref, retr_ref, ts_ref, cons_ref, conf_ref, s_ref, sem):
    rows = PADTOT // 128
    s_ref[...] = scores_ref[...].reshape(rows, 128)
    flat = (lax.broadcasted_iota(jnp.int32, (rows, 128), 0) * 128
            + lax.broadcasted_iota(jnp.int32, (rows, 128), 1))
    for i in range(5):
        s = s_ref[...]
        m = jnp.max(s)
        slot = jnp.min(jnp.where(s == m, flat, PADTOT))
        ts_ref[i] = m
        key = (slot >> 12) * RB + (slot & (PB - 1))
        cp = pltpu.make_async_copy(values_ref.at[pl.ds(key, 1), :],
                                   retr_ref.at[pl.ds(i, 1), :], sem)
        cp.start()
        cp.wait()
        s_ref[...] = jnp.where(flat == slot, NEG, s)
    ev = retr_ref[...]
    A = V1_ref[:, 0:D]
    Bm = V1_ref[:, D:2 * D]
    ca = lax.dot_general(claim_ref[...], A, (((1,), (1,)), ((), ())),
                         preferred_element_type=jnp.float32)
    eb = lax.dot_general(ev, Bm, (((1,), (1,)), ((), ())),
                         preferred_element_type=jnp.float32)
    E1 = (lax.broadcasted_iota(jnp.int32, (20, 4), 0) // 5
          == lax.broadcasted_iota(jnp.int32, (20, 4), 1)).astype(jnp.float32)
    E2 = (lax.broadcasted_iota(jnp.int32, (20, 5), 0) % 5
          == lax.broadcasted_iota(jnp.int32, (20, 5), 1)).astype(jnp.float32)
    pairsum = (jnp.dot(E1, ca, preferred_element_type=jnp.float32)
               + jnp.dot(E2, eb, preferred_element_type=jnp.float32)
               + c1_ref[...][None, :])
    h = jnp.maximum(pairsum, 0.0)
    logits = lax.dot_general(h, V2_ref[...], (((1,), (1,)), ((), ())),
                             preferred_element_type=jnp.float32)
    E3 = (lax.broadcasted_iota(jnp.int32, (20, 5), 0) % 5
          == lax.broadcasted_iota(jnp.int32, (20, 5), 1)).astype(jnp.float32)
    E4 = (lax.broadcasted_iota(jnp.int32, (4, 20), 0)
          == lax.broadcasted_iota(jnp.int32, (4, 20), 1) // 5).astype(jnp.float32)
    l45 = jnp.dot(E4, logits * E3, preferred_element_type=jnp.float32)
    cons = jax.nn.sigmoid(l45 + c2_ref[0])
    cons_ref[...] = cons
    conf_ref[0] = jnp.max(cons)


def kernel(claim_embedding, keys, values, Wq, bq, Wk, bk, V1, c1, V2, c2):
    WqT = Wq.T
    WkT = Wk.T
    scores = pl.pallas_call(
        _score_body,
        grid=(NBLK,),
        in_specs=[
            pl.BlockSpec((4, D), lambda i: (0, 0)),
            pl.BlockSpec((D, D), lambda i: (0, 0)),
            pl.BlockSpec((D,), lambda i: (0,)),
            pl.BlockSpec((D, D), lambda i: (0, 0)),
            pl.BlockSpec((D,), lambda i: (0,)),
            pl.BlockSpec((RB, D), lambda i: (i, 0)),
        ],
        out_specs=pl.BlockSpec((PB,), lambda i: (i,)),
        out_shape=jax.ShapeDtypeStruct((PADTOT,), jnp.float32),
    )(claim_embedding, WqT, bq, WkT, bk, keys)

    retr, ts, cons, conf = pl.pallas_call(
        _select_body,
        in_specs=[
            pl.BlockSpec((PADTOT,), lambda: (0,)),
            pl.BlockSpec(memory_space=pl.ANY),
            pl.BlockSpec((4, D), lambda: (0, 0)),
            pl.BlockSpec((D, 2 * D), lambda: (0, 0)),
            pl.BlockSpec((D,), lambda: (0,)),
            pl.BlockSpec((1, D), lambda: (0, 0)),
            pl.BlockSpec(memory_space=pltpu.SMEM),
        ],
        out_specs=[
            pl.BlockSpec((5, D), lambda: (0, 0)),
            pl.BlockSpec(memory_space=pltpu.SMEM),
            pl.BlockSpec((4, 5), lambda: (0, 0)),
            pl.BlockSpec(memory_space=pltpu.SMEM),
        ],
        out_shape=[
            jax.ShapeDtypeStruct((5, D), jnp.float32),
            jax.ShapeDtypeStruct((5,), jnp.float32),
            jax.ShapeDtypeStruct((4, 5), jnp.float32),
            jax.ShapeDtypeStruct((1,), jnp.float32),
        ],
        scratch_shapes=[
            pltpu.VMEM((PADTOT // 128, 128), jnp.float32),
            pltpu.SemaphoreType.DMA,
        ],
    )(scores, values, claim_embedding, V1, c1, V2, c2)

    return (retr, ts, cons, conf[0])


# speedup vs baseline: 3.0298x; 3.0298x over previous
import jax
import jax.numpy as jnp
from jax import lax
from jax.experimental import pallas as pl
from jax.experimental.pallas import tpu as pltpu

D = 256
KEV = 100000
PB = 4096
NBLK = -(-KEV // PB)
PADTOT = NBLK * PB
NEG = float("-inf")


def _score_body(claim_ref, Wq_ref, bq_ref, Wk_ref, bkc_ref,
                keys_ref, out_ref):
    c0 = claim_ref[0:1, :]
    qr = lax.dot_general(c0, Wq_ref[...], (((1,), (1,)), ((), ())),
                         preferred_element_type=jnp.float32)
    qr = qr + bq_ref[...][None, :]
    qn = qr / jnp.maximum(jnp.sqrt(jnp.sum(qr * qr)), 1e-12)
    zT = lax.dot_general(Wk_ref[...], keys_ref[...], (((1,), (1,)), ((), ())),
                         preferred_element_type=jnp.float32)
    zT = zT + bkc_ref[...]
    num = lax.dot_general(qn, zT, (((1,), (0,)), ((), ())),
                          preferred_element_type=jnp.float32)
    d2 = jnp.sum(zT * zT, axis=0, keepdims=True)
    s = num / jnp.maximum(jnp.sqrt(d2), 1e-12)
    gid = (lax.broadcasted_iota(jnp.int32, (1, PB), 1)
           + pl.program_id(0) * PB)
    out_ref[...] = jnp.where(gid < KEV, s, NEG).reshape(PB)


def _select_body(scores_ref, values_ref, claim_ref, V1_ref, c1_ref, V2_ref,
                 c2_ref, retr_ref, ts_ref, cons_ref, conf_ref, s_ref, sem):
    rows = PADTOT // 128
    s_ref[...] = scores_ref[...].reshape(rows, 128)
    flat = (lax.broadcasted_iota(jnp.int32, (rows, 128), 0) * 128
            + lax.broadcasted_iota(jnp.int32, (rows, 128), 1))
    for i in range(5):
        s = s_ref[...]
        m = jnp.max(s)
        slot = jnp.min(jnp.where(s == m, flat, PADTOT))
        ts_ref[i] = m
        key = slot
        cp = pltpu.make_async_copy(values_ref.at[pl.ds(key, 1), :],
                                   retr_ref.at[pl.ds(i, 1), :], sem)
        cp.start()
        cp.wait()
        s_ref[...] = jnp.where(flat == slot, NEG, s)
    ev = retr_ref[...]
    A = V1_ref[:, 0:D]
    Bm = V1_ref[:, D:2 * D]
    ca = lax.dot_general(claim_ref[...], A, (((1,), (1,)), ((), ())),
                         preferred_element_type=jnp.float32)
    eb = lax.dot_general(ev, Bm, (((1,), (1,)), ((), ())),
                         preferred_element_type=jnp.float32)
    E1 = (lax.broadcasted_iota(jnp.int32, (20, 4), 0) // 5
          == lax.broadcasted_iota(jnp.int32, (20, 4), 1)).astype(jnp.float32)
    E2 = (lax.broadcasted_iota(jnp.int32, (20, 5), 0) % 5
          == lax.broadcasted_iota(jnp.int32, (20, 5), 1)).astype(jnp.float32)
    pairsum = (jnp.dot(E1, ca, preferred_element_type=jnp.float32)
               + jnp.dot(E2, eb, preferred_element_type=jnp.float32)
               + c1_ref[...][None, :])
    h = jnp.maximum(pairsum, 0.0)
    logits = lax.dot_general(h, V2_ref[...], (((1,), (1,)), ((), ())),
                             preferred_element_type=jnp.float32)
    E3 = (lax.broadcasted_iota(jnp.int32, (20, 5), 0) % 5
          == lax.broadcasted_iota(jnp.int32, (20, 5), 1)).astype(jnp.float32)
    E4 = (lax.broadcasted_iota(jnp.int32, (4, 20), 0)
          == lax.broadcasted_iota(jnp.int32, (4, 20), 1) // 5).astype(jnp.float32)
    l45 = jnp.dot(E4, logits * E3, preferred_element_type=jnp.float32)
    cons = jax.nn.sigmoid(l45 + c2_ref[0])
    cons_ref[...] = cons
    conf_ref[0] = jnp.max(cons)


def kernel(claim_embedding, keys, values, Wq, bq, Wk, bk, V1, c1, V2, c2):
    scores = pl.pallas_call(
        _score_body,
        grid=(NBLK,),
        in_specs=[
            pl.BlockSpec((4, D), lambda i: (0, 0)),
            pl.BlockSpec((D, D), lambda i: (0, 0)),
            pl.BlockSpec((D,), lambda i: (0,)),
            pl.BlockSpec((D, D), lambda i: (0, 0)),
            pl.BlockSpec((D, 1), lambda i: (0, 0)),
            pl.BlockSpec((PB, D), lambda i: (i, 0)),
        ],
        out_specs=pl.BlockSpec((PB,), lambda i: (i,)),
        out_shape=jax.ShapeDtypeStruct((PADTOT,), jnp.float32),
    )(claim_embedding, Wq, bq, Wk, bk.reshape(D, 1), keys)

    retr, ts, cons, conf = pl.pallas_call(
        _select_body,
        in_specs=[
            pl.BlockSpec((PADTOT,), lambda: (0,)),
            pl.BlockSpec(memory_space=pl.ANY),
            pl.BlockSpec((4, D), lambda: (0, 0)),
            pl.BlockSpec((D, 2 * D), lambda: (0, 0)),
            pl.BlockSpec((D,), lambda: (0,)),
            pl.BlockSpec((1, D), lambda: (0, 0)),
            pl.BlockSpec(memory_space=pltpu.SMEM),
        ],
        out_specs=[
            pl.BlockSpec((5, D), lambda: (0, 0)),
            pl.BlockSpec(memory_space=pltpu.SMEM),
            pl.BlockSpec((4, 5), lambda: (0, 0)),
            pl.BlockSpec(memory_space=pltpu.SMEM),
        ],
        out_shape=[
            jax.ShapeDtypeStruct((5, D), jnp.float32),
            jax.ShapeDtypeStruct((5,), jnp.float32),
            jax.ShapeDtypeStruct((4, 5), jnp.float32),
            jax.ShapeDtypeStruct((1,), jnp.float32),
        ],
        scratch_shapes=[
            pltpu.VMEM((PADTOT // 128, 128), jnp.float32),
            pltpu.SemaphoreType.DMA,
        ],
    )(scores, values, claim_embedding, V1, c1, V2, c2)

    return (retr, ts, cons, conf[0])
